# Initial kernel scaffold; baseline (speedup 1.0000x reference)
import jax
import jax.numpy as jnp
from jax.experimental import pallas as pl


def kernel(x, edge_index, edge_attr, W, W_edge, att, W_proj, b_proj):
    raise NotImplementedError("write your pallas kernel here")


# speedup vs baseline: 17.1853x; 17.1853x over previous
import dataclasses
import functools

import jax
import jax.numpy as jnp
from jax import lax
from jax.experimental import pallas as pl
from jax.experimental.pallas import tpu as pltpu
from jax.experimental.pallas import tpu_sc as plsc

N = 10000
E = 320000
IN_DIM = 128
HID = 32
OUT_DIM = 128
H = 4
EDGE_DIM = 16
NEG = 0.2

NC = 2
NS = 16
NT = NC * NS
EPT = E // NT
K = 80
NPT = N // NS

EB = 8000


def _leaky(v):
    return jnp.maximum(v, NEG * v)


def _prep_nodes_body(x_ref, wcat_ref, aw_ref, xw_ref, atab_ref, m_ref):
    xw = jnp.dot(x_ref[...], wcat_ref[...], preferred_element_type=jnp.float32)
    xw_ref[...] = xw
    a = jnp.dot(xw, aw_ref[...], preferred_element_type=jnp.float32)
    atab_ref[...] = a
    amax = jnp.max(a, axis=0)
    m_ref[...] = jnp.broadcast_to(amax[:, None], (2 * H, 16))


def _prep_nodes(x, w_cat, aw):
    return pl.pallas_call(
        _prep_nodes_body,
        out_shape=[
            jax.ShapeDtypeStruct((N, H * HID), jnp.float32),
            jax.ShapeDtypeStruct((N, 2 * H), jnp.float32),
            jax.ShapeDtypeStruct((2 * H, 16), jnp.float32),
        ],
    )(x, w_cat, aw)


def _prep_edges_body(ea_ref, wb_ref, b_ref, mb_ref):
    i = pl.program_id(0)
    blk = jnp.dot(ea_ref[...], wb_ref[...], preferred_element_type=jnp.float32)
    b_ref[...] = blk
    rep = jnp.broadcast_to(jnp.max(blk, axis=0)[:, None], (H, 16))

    @pl.when(i == 0)
    def _():
        mb_ref[...] = rep

    @pl.when(i > 0)
    def _():
        mb_ref[...] = jnp.maximum(mb_ref[...], rep)


def _prep_edges(ea, wb):
    return pl.pallas_call(
        _prep_edges_body,
        grid=(E // EB,),
        in_specs=[
            pl.BlockSpec((EB, EDGE_DIM), lambda i: (i, 0)),
            pl.BlockSpec((EDGE_DIM, H), lambda i: (0, 0)),
        ],
        out_specs=[
            pl.BlockSpec((EB, H), lambda i: (i, 0)),
            pl.BlockSpec((H, 16), lambda i: (0, 0)),
        ],
        out_shape=[
            jax.ShapeDtypeStruct((E, H), jnp.float32),
            jax.ShapeDtypeStruct((H, 16), jnp.float32),
        ],
    )(ea, wb)


ROW = 144


def _sc_compiler_params():
    cp = pltpu.CompilerParams()
    fields = pltpu.CompilerParams.__dataclass_fields__
    if "needs_layout_passes" in fields:
        cp = dataclasses.replace(cp, needs_layout_passes=False)
    if "use_tc_tiling_on_sc" in fields:
        cp = dataclasses.replace(cp, use_tc_tiling_on_sc=False)
    return cp


def _sc_gat(src, dst, xw, atab, b, mam, mb):
    mesh = plsc.VectorSubcoreMesh(core_axis_name="c", subcore_axis_name="s")

    @functools.partial(
        pl.kernel,
        out_type=jax.ShapeDtypeStruct((NC, N, ROW), jnp.float32),
        mesh=mesh,
        compiler_params=_sc_compiler_params(),
        scratch_types=[
            pltpu.VMEM((K,), jnp.int32),
            pltpu.VMEM((K,), jnp.int32),
            pltpu.VMEM((K, H), jnp.float32),
            pltpu.VMEM((K, 2 * H), jnp.float32),
            pltpu.VMEM((K, 2 * H), jnp.float32),
            pltpu.VMEM((K, H * HID), jnp.float32),
            pltpu.VMEM((K, ROW), jnp.float32),
            pltpu.VMEM((2 * H, 16), jnp.float32),
            pltpu.VMEM((H, 16), jnp.float32),
            pltpu.VMEM_SHARED((N, ROW), jnp.float32),
        ],
    )
    def body(src_hbm, dst_hbm, xw_hbm, a_hbm, b_hbm, mam_hbm, mb_hbm,
             oacc_hbm,
             src_v, dst_v, b_v, ai_v, aj_v, xw_v, msg_v, mam_v, mb_v, out_sp):
        c = lax.axis_index("c")
        sub = lax.axis_index("s")
        tid = c * NS + sub
        ebase = tid * EPT
        nbase = sub * NPT

        pltpu.sync_copy(mam_hbm, mam_v)
        pltpu.sync_copy(mb_hbm, mb_v)

        mh = []
        for h in range(H):
            mh.append(_leaky(mam_v[h, :] + mam_v[H + h, :] + mb_v[h, :]))

        zero16 = jnp.zeros((16,), jnp.float32)

        @pl.loop(0, K)
        def _(k):
            for j in range(ROW // 16):
                msg_v[k, pl.ds(j * 16, 16)] = zero16

        for i in range(NPT // K):
            pltpu.sync_copy(msg_v, out_sp.at[pl.ds(nbase + i * K, K)])
        rem = NPT - (NPT // K) * K
        pltpu.sync_copy(msg_v.at[pl.ds(0, rem)],
                        out_sp.at[pl.ds(nbase + NPT - rem, rem)])
        plsc.subcore_barrier()

        iota16 = lax.iota(jnp.int32, 16)

        @pl.loop(0, EPT // K)
        def _(ci):
            off = ebase + ci * K
            pltpu.sync_copy(src_hbm.at[pl.ds(off, K)], src_v)
            pltpu.sync_copy(dst_hbm.at[pl.ds(off, K)], dst_v)
            pltpu.sync_copy(b_hbm.at[pl.ds(off, K)], b_v)
            pltpu.sync_copy(xw_hbm.at[src_v], xw_v)
            pltpu.sync_copy(a_hbm.at[dst_v], ai_v)
            pltpu.sync_copy(a_hbm.at[src_v], aj_v)

            for g in range(K // 16):
                ridx = iota16 + (g * 16)
                for h in range(H):
                    hfull = jnp.full((16,), h, jnp.int32)
                    ai = plsc.load_gather(ai_v, [ridx, hfull])
                    aj = plsc.load_gather(aj_v, [ridx, jnp.full((16,), H + h, jnp.int32)])
                    bh = plsc.load_gather(b_v, [ridx, hfull])
                    exh = jnp.exp(_leaky(ai + aj + bh) - mh[h])
                    plsc.store_scatter(msg_v, [ridx, hfull + (H * HID)], exh)

            @pl.loop(0, K)
            def _(k):
                ex16 = msg_v[k, pl.ds(H * HID, 16)]
                for h in range(H):
                    exs = ex16[h]
                    for q in range(2):
                        col = h * HID + q * 16
                        msg_v[k, pl.ds(col, 16)] = xw_v[k, pl.ds(col, 16)] * exs

            pltpu.sync_copy(msg_v, out_sp.at[dst_v], add=True)

        plsc.subcore_barrier()
        pltpu.sync_copy(out_sp.at[pl.ds(nbase, NPT)],
                        oacc_hbm.at[c, pl.ds(nbase, NPT)])

    return body(src, dst, xw, atab, b, mam, mb)


def _post_body(oa_ref, r_ref, wp_ref, bp_ref, o_ref):
    acc = oa_ref[0] + oa_ref[1]
    o = acc[:, :H * HID]
    s = acc[:, H * HID:H * HID + H]
    srep = jnp.dot(s, r_ref[...], preferred_element_type=jnp.float32)
    y = o / (srep + 1e-16)
    z = jnp.dot(y, wp_ref[...], preferred_element_type=jnp.float32) + bp_ref[...]
    o_ref[...] = jnp.where(z > 0, z, jnp.exp(z) - 1.0)


def _post(oacc, r4, w_proj, b_proj2):
    return pl.pallas_call(
        _post_body,
        out_shape=jax.ShapeDtypeStruct((N, OUT_DIM), jnp.float32),
    )(oacc, r4, w_proj, b_proj2)


def kernel(x, edge_index, edge_attr, W, W_edge, att, W_proj, b_proj):
    src = edge_index[0].astype(jnp.int32)
    dst = edge_index[1].astype(jnp.int32)

    w_cat = W.transpose(1, 0, 2).reshape(IN_DIM, H * HID)
    aw = jnp.zeros((H * HID, 2 * H), jnp.float32)
    for h in range(H):
        aw = aw.at[h * HID:(h + 1) * HID, h].set(att[h, :HID, 0])
        aw = aw.at[h * HID:(h + 1) * HID, H + h].set(att[h, HID:2 * HID, 0])
    wb = jnp.stack([W_edge[h] @ att[h, 2 * HID:, 0] for h in range(H)], axis=1)
    r4 = jnp.kron(jnp.eye(H, dtype=jnp.float32),
                  jnp.ones((1, HID), jnp.float32))
    b_proj2 = b_proj.reshape(1, OUT_DIM)

    xw, atab, mam = _prep_nodes(x, w_cat, aw)
    b, mb = _prep_edges(edge_attr, wb)
    oacc = _sc_gat(src, dst, xw, atab, b, mam, mb)
    return _post(oacc, r4, W_proj, b_proj2)
